# baseline (speedup 1.0000x reference)
import functools

import jax
import jax.numpy as jnp
from jax import lax
from jax.experimental import pallas as pl
from jax.experimental.pallas import tpu as pltpu
from jax.experimental.pallas import tpu_sc as plsc

_D = 32
_L = 16
_NC = 2
_NS = 16
_NW = _NC * _NS
_NB = 8


@functools.lru_cache(maxsize=None)
def _gather_call(bsz, seq):
    bpw = bsz // _NW
    ng = bpw // _NB
    ngr = -(-seq // _L)
    seq_pad = ngr * _L
    segs = []
    off = 0
    while off < seq:
        n = min(128, seq - off)
        segs.append((off, n))
        off += n
    mesh = plsc.VectorSubcoreMesh(core_axis_name="c", subcore_axis_name="s")

    @functools.partial(
        pl.kernel,
        mesh=mesh,
        out_type=jax.ShapeDtypeStruct((bsz, seq, _D), jnp.float32),
        scratch_types=[
            pltpu.VMEM((seq, bpw), jnp.int32),
            pltpu.VMEM((_NB, seq_pad), jnp.int32),
            pltpu.VMEM((2, _NB, seq, _D), jnp.float32),
            pltpu.SemaphoreType.DMA,
            (pltpu.SemaphoreType.DMA, pltpu.SemaphoreType.DMA),
        ],
        compiler_params=pltpu.CompilerParams(
            use_tc_tiling_on_sc=False, needs_layout_passes=False),
    )
    def k(idxt_hbm, tab_hbm, out_hbm, idx_sv, idx_bv, rows_v, gsem, wsems):
        wid = lax.axis_index("s") * _NC + lax.axis_index("c")
        brow0 = wid * bpw
        pltpu.sync_copy(idxt_hbm.at[:, pl.ds(brow0, bpw)], idx_sv)

        def wb_wait(b):
            pltpu.make_async_copy(
                rows_v.at[b], out_hbm.at[pl.ds(0, _NB)], wsems[b]).wait()

        def do_chunk(g, b):
            for r in range(_NB):
                bcol = g * _NB + r
                bvec = jnp.full((_L,), bcol, jnp.int32)
                for t in range(ngr):
                    svec = jnp.minimum(
                        lax.iota(jnp.int32, _L) + t * _L, seq - 1)
                    vals = plsc.load_gather(idx_sv, [svec, bvec])
                    idx_bv[r, pl.ds(t * _L, _L)] = vals
            copies = [
                pltpu.async_copy(
                    tab_hbm.at[idx_bv.at[r, pl.ds(soff, slen)]],
                    rows_v.at[b, r, pl.ds(soff, slen)],
                    gsem,
                )
                for r in range(_NB)
                for soff, slen in segs
            ]
            for c in copies:
                c.wait()
            pltpu.make_async_copy(
                rows_v.at[b],
                out_hbm.at[pl.ds(brow0 + g * _NB, _NB)],
                wsems[b],
            ).start()

        def body(g2, carry):
            g = g2 * 2

            @pl.when(g2 > 0)
            def _():
                wb_wait(0)

            do_chunk(g, 0)

            @pl.when(g2 > 0)
            def _():
                wb_wait(1)

            do_chunk(g + 1, 1)
            return carry

        lax.fori_loop(0, ng // 2, body, 0)
        wb_wait(0)
        wb_wait(1)

    return k


def kernel(x, w):
    bsz, seq = x.shape
    assert bsz % (_NW * 2 * _NB) == 0 and seq % 8 == 0
    xt = jnp.swapaxes(x.astype(jnp.int32), 0, 1)
    return _gather_call(bsz, seq)(xt, w)
